# Initial kernel scaffold; baseline (speedup 1.0000x reference)
import jax
import jax.numpy as jnp
from jax.experimental import pallas as pl


def kernel(input_ids, attn_mask, token_table, pos_table):
    raise NotImplementedError("write your pallas kernel here")


# speedup vs baseline: 2.8215x; 2.8215x over previous
import functools

import jax
import jax.numpy as jnp
from jax import lax
from jax.experimental import pallas as pl
from jax.experimental.pallas import tpu as pltpu
from jax.experimental.pallas import tpu_sc as plsc

B = 1024
T = 1024
EMB = 64

NC = 2
NS = 16
NB = 4
NT = 8
BBLK = B // NB
TBLK = T // NT


def _emb_body(idx_hbm, table_hbm, pos_hbm, out_hbm, idx_v, pos_v, rows_v, sem):
    c = lax.axis_index("c")
    s = lax.axis_index("s")
    wid = s * NC + c
    bb = wid // NT
    tb = wid % NT
    b0 = bb * BBLK
    t0 = tb * TBLK

    pltpu.sync_copy(pos_hbm.at[pl.ds(t0, TBLK)], pos_v)
    pltpu.sync_copy(idx_hbm.at[pl.ds(b0, BBLK), pl.ds(t0, TBLK)], idx_v)

    @pl.loop(0, BBLK)
    def _row(b):
        pltpu.async_copy(table_hbm.at[idx_v.at[b]], rows_v, sem).wait()

        @pl.loop(0, TBLK)
        def _add(r):
            for g in range(EMB // 16):
                sl = pl.ds(g * 16, 16)
                rows_v[r, sl] = rows_v[r, sl] + pos_v[r, sl]

        pltpu.sync_copy(rows_v, out_hbm.at[b0 + b, pl.ds(t0, TBLK)])


@jax.jit
def _emb(input_ids, token_table, pos_table):
    mesh = plsc.VectorSubcoreMesh(core_axis_name="c", subcore_axis_name="s")
    f = pl.kernel(
        _emb_body,
        out_type=jax.ShapeDtypeStruct((B, T, EMB), jnp.float32),
        mesh=mesh,
        scratch_types=[
            pltpu.VMEM((BBLK, TBLK), jnp.int32),
            pltpu.VMEM((TBLK, EMB), jnp.float32),
            pltpu.VMEM((TBLK, EMB), jnp.float32),
            pltpu.SemaphoreType.DMA,
        ],
        compiler_params=pltpu.CompilerParams(use_tc_tiling_on_sc=False),
    )
    return f(input_ids, token_table, pos_table)


def kernel(input_ids, attn_mask, token_table, pos_table):
    out = _emb(input_ids.astype(jnp.int32), token_table, pos_table)
    return (out, attn_mask)


# speedup vs baseline: 2.9938x; 1.0611x over previous
import jax
import jax.numpy as jnp
from jax import lax
from jax.experimental import pallas as pl
from jax.experimental.pallas import tpu as pltpu
from jax.experimental.pallas import tpu_sc as plsc

B = 1024
T = 1024
EMB = 64

NC = 2
NS = 16
NB = 4
NT = 8
BBLK = B // NB
TBLK = T // NT
NBUF = 4


def _emb_body(idx_hbm, table_hbm, pos_hbm, out_hbm, idx_v, pos_v, rows_v,
              g0, g1, g2, g3, w0, w1, w2, w3):
    gs = [g0, g1, g2, g3]
    ws = [w0, w1, w2, w3]
    c = lax.axis_index("c")
    s = lax.axis_index("s")
    wid = s * NC + c
    bb = wid // NT
    tb = wid % NT
    b0 = bb * BBLK
    t0 = tb * TBLK

    pltpu.sync_copy(pos_hbm.at[pl.ds(t0, TBLK)], pos_v)
    pltpu.sync_copy(idx_hbm.at[pl.ds(b0, BBLK), pl.ds(t0, TBLK)], idx_v)

    def gather(j, k):
        pltpu.make_async_copy(table_hbm.at[idx_v.at[j]], rows_v.at[k],
                              gs[k]).start()

    def wait_gather(k):
        pltpu.make_async_copy(table_hbm.at[idx_v.at[0]], rows_v.at[k],
                              gs[k]).wait()

    def write(j, k):
        pltpu.make_async_copy(rows_v.at[k],
                              out_hbm.at[b0 + j, pl.ds(t0, TBLK)],
                              ws[k]).start()

    def wait_write(k):
        pltpu.make_async_copy(rows_v.at[k],
                              out_hbm.at[b0, pl.ds(t0, TBLK)],
                              ws[k]).wait()

    def add(k):
        @pl.loop(0, TBLK, unroll=4)
        def _add(r):
            for g in range(EMB // 16):
                sl = pl.ds(g * 16, 16)
                rows_v[k, r, sl] = rows_v[k, r, sl] + pos_v[r, sl]

    gather(0, 0)
    gather(1, 1)
    for j in range(2):
        wait_gather(j)
        add(j)
        write(j, j)
        gather(j + 2, j + 2)

    @pl.loop(0, (BBLK - 4) // 4)
    def _grp(q):
        for i in range(4):
            j = q * 4 + 2 + i
            k = (2 + i) % NBUF
            wait_gather(k)
            add(k)
            write(j, k)
            k2 = (k + 2) % NBUF
            wait_write(k2)
            gather(j + 2, k2)

    for j in range(BBLK - 2, BBLK):
        k = j % NBUF
        wait_gather(k)
        add(k)
        write(j, k)
    for k in range(NBUF):
        wait_write(k)


@jax.jit
def _emb(input_ids, token_table, pos_table):
    mesh = plsc.VectorSubcoreMesh(core_axis_name="c", subcore_axis_name="s")
    f = pl.kernel(
        _emb_body,
        out_type=jax.ShapeDtypeStruct((B, T, EMB), jnp.float32),
        mesh=mesh,
        scratch_types=[
            pltpu.VMEM((BBLK, TBLK), jnp.int32),
            pltpu.VMEM((TBLK, EMB), jnp.float32),
            pltpu.VMEM((NBUF, TBLK, EMB), jnp.float32),
        ] + [pltpu.SemaphoreType.DMA] * (2 * NBUF),
        compiler_params=pltpu.CompilerParams(use_tc_tiling_on_sc=False),
    )
    return f(input_ids, token_table, pos_table)


def kernel(input_ids, attn_mask, token_table, pos_table):
    out = _emb(input_ids.astype(jnp.int32), token_table, pos_table)
    return (out, attn_mask)


# speedup vs baseline: 3.2940x; 1.1003x over previous
import jax
import jax.numpy as jnp
from jax import lax
from jax.experimental import pallas as pl
from jax.experimental.pallas import tpu as pltpu
from jax.experimental.pallas import tpu_sc as plsc

B = 1024
T = 1024
EMB = 64

NC = 2
NS = 16
NB = 4
NT = 8
BBLK = B // NB
TBLK = T // NT
NBUF = 8
AHEAD = NBUF - 2


def _emb_body(idx_hbm, table_hbm, pos_hbm, out_hbm, idx_v, pos_v, rows_v,
              *sems):
    gs = sems[:NBUF]
    ws = sems[NBUF:]
    c = lax.axis_index("c")
    s = lax.axis_index("s")
    wid = s * NC + c
    bb = wid // NT
    tb = wid % NT
    b0 = bb * BBLK
    t0 = tb * TBLK

    pltpu.sync_copy(pos_hbm.at[pl.ds(t0, TBLK)], pos_v)
    pltpu.sync_copy(idx_hbm.at[pl.ds(b0, BBLK), pl.ds(t0, TBLK)], idx_v)

    def gather(j, k):
        pltpu.make_async_copy(table_hbm.at[idx_v.at[j]], rows_v.at[k],
                              gs[k]).start()

    def wait_gather(k):
        pltpu.make_async_copy(table_hbm.at[idx_v.at[0]], rows_v.at[k],
                              gs[k]).wait()

    def write(j, k):
        pltpu.make_async_copy(rows_v.at[k],
                              out_hbm.at[b0 + j, pl.ds(t0, TBLK)],
                              ws[k]).start()

    def wait_write(k):
        pltpu.make_async_copy(rows_v.at[k],
                              out_hbm.at[b0, pl.ds(t0, TBLK)],
                              ws[k]).wait()

    def add(k):
        @pl.loop(0, TBLK, unroll=4)
        def _add(r):
            for g in range(EMB // 16):
                sl = pl.ds(g * 16, 16)
                rows_v[k, r, sl] = rows_v[k, r, sl] + pos_v[r, sl]

    for j in range(AHEAD):
        gather(j, j)
    for j in range(2):
        wait_gather(j)
        add(j)
        write(j, j)
        gather(j + AHEAD, (j + AHEAD) % NBUF)

    NSTEADY = (BBLK - AHEAD - 2) // NBUF

    @pl.loop(0, NSTEADY)
    def _grp(q):
        for i in range(NBUF):
            j = q * NBUF + 2 + i
            k = (2 + i) % NBUF
            wait_gather(k)
            add(k)
            write(j, k)
            k2 = (k + AHEAD) % NBUF
            wait_write(k2)
            gather(j + AHEAD, k2)

    for j in range(BBLK - AHEAD, BBLK):
        k = j % NBUF
        wait_gather(k)
        add(k)
        write(j, k)
    for k in range(NBUF):
        wait_write(k)


@jax.jit
def _emb(input_ids, token_table, pos_table):
    mesh = plsc.VectorSubcoreMesh(core_axis_name="c", subcore_axis_name="s")
    f = pl.kernel(
        _emb_body,
        out_type=jax.ShapeDtypeStruct((B, T, EMB), jnp.float32),
        mesh=mesh,
        scratch_types=[
            pltpu.VMEM((BBLK, TBLK), jnp.int32),
            pltpu.VMEM((TBLK, EMB), jnp.float32),
            pltpu.VMEM((NBUF, TBLK, EMB), jnp.float32),
        ] + [pltpu.SemaphoreType.DMA] * (2 * NBUF),
        compiler_params=pltpu.CompilerParams(use_tc_tiling_on_sc=False),
    )
    return f(input_ids, token_table, pos_table)


def kernel(input_ids, attn_mask, token_table, pos_table):
    out = _emb(input_ids.astype(jnp.int32), token_table, pos_table)
    return (out, attn_mask)


# speedup vs baseline: 3.5050x; 1.0641x over previous
import jax
import jax.numpy as jnp
from jax import lax
from jax.experimental import pallas as pl
from jax.experimental.pallas import tpu as pltpu
from jax.experimental.pallas import tpu_sc as plsc

B = 1024
T = 1024
EMB = 64

NC = 2
NS = 16
NB = 8
NT = 4
BBLK = B // NB
TBLK = T // NT
NBUF = 8
AHEAD = NBUF - 2


def _emb_body(idxt_hbm, table_hbm, pos_hbm, out_hbm, idx_v, pos_v, rows_v,
              *sems):
    gs = sems[:NBUF]
    ws = sems[NBUF:]
    c = lax.axis_index("c")
    s = lax.axis_index("s")
    wid = s * NC + c
    bb = wid // NT
    tb = wid % NT
    b0 = bb * BBLK
    t0 = tb * TBLK

    pltpu.sync_copy(pos_hbm.at[pl.ds(t0, TBLK)], pos_v)
    pltpu.sync_copy(idxt_hbm.at[pl.ds(t0, TBLK), pl.ds(b0, BBLK)], idx_v)

    def gather(t, k):
        pltpu.make_async_copy(table_hbm.at[idx_v.at[t]], rows_v.at[k],
                              gs[k]).start()

    def wait_gather(k):
        pltpu.make_async_copy(table_hbm.at[idx_v.at[0]], rows_v.at[k],
                              gs[k]).wait()

    def write(t, k):
        pltpu.make_async_copy(rows_v.at[k],
                              out_hbm.at[pl.ds(b0, BBLK), t0 + t],
                              ws[k]).start()

    def wait_write(k):
        pltpu.make_async_copy(rows_v.at[k],
                              out_hbm.at[pl.ds(b0, BBLK), t0],
                              ws[k]).wait()

    def add(t, k):
        pv = [pos_v[t, pl.ds(g * 16, 16)] for g in range(EMB // 16)]

        @pl.loop(0, BBLK, unroll=4)
        def _add(r):
            for g in range(EMB // 16):
                sl = pl.ds(g * 16, 16)
                rows_v[k, r, sl] = rows_v[k, r, sl] + pv[g]

    for t in range(AHEAD):
        gather(t, t)
    for t in range(2):
        wait_gather(t)
        add(t, t)
        write(t, t)
        gather(t + AHEAD, (t + AHEAD) % NBUF)

    NSTEADY = (TBLK - AHEAD - 2) // NBUF

    @pl.loop(0, NSTEADY)
    def _grp(q):
        for i in range(NBUF):
            t = q * NBUF + 2 + i
            k = (2 + i) % NBUF
            wait_gather(k)
            add(t, k)
            write(t, k)
            k2 = (k + AHEAD) % NBUF
            wait_write(k2)
            gather(t + AHEAD, k2)

    for t in range(TBLK - AHEAD, TBLK):
        k = t % NBUF
        wait_gather(k)
        add(t, k)
        write(t, k)
    for k in range(NBUF):
        wait_write(k)


@jax.jit
def _emb(input_ids_t, token_table, pos_table):
    mesh = plsc.VectorSubcoreMesh(core_axis_name="c", subcore_axis_name="s")
    f = pl.kernel(
        _emb_body,
        out_type=jax.ShapeDtypeStruct((B, T, EMB), jnp.float32),
        mesh=mesh,
        scratch_types=[
            pltpu.VMEM((TBLK, BBLK), jnp.int32),
            pltpu.VMEM((TBLK, EMB), jnp.float32),
            pltpu.VMEM((NBUF, BBLK, EMB), jnp.float32),
        ] + [pltpu.SemaphoreType.DMA] * (2 * NBUF),
        compiler_params=pltpu.CompilerParams(use_tc_tiling_on_sc=False),
    )
    return f(input_ids_t, token_table, pos_table)


def kernel(input_ids, attn_mask, token_table, pos_table):
    idx_t = input_ids.astype(jnp.int32).T
    out = _emb(idx_t, token_table, pos_table)
    return (out, attn_mask)
